# Initial kernel scaffold; baseline (speedup 1.0000x reference)
import jax
import jax.numpy as jnp
from jax.experimental import pallas as pl


def kernel(x, edge_index, e, xbatch, bn_node_g, bn_node_b, bn_edge_g, bn_edge_b, nn0_w1, nn0_b1, nn0_w2, nn0_b2, root0, bias0, nn1_w1, nn1_b1, nn1_w2, nn1_b2, root1, bias1, nn2_w1, nn2_b1, nn2_w2, nn2_b2, root2, bias2, em_w1, em_b1, em_w2, em_b2, em_w3, em_b3, nm_w1, nm_b1, nm_w2, nm_b2):
    raise NotImplementedError("write your pallas kernel here")


# speedup vs baseline: 1.5067x; 1.5067x over previous
import functools

import numpy as np
import jax
import jax.numpy as jnp
from jax import lax
from jax.experimental import pallas as pl
from jax.experimental.pallas import tpu as pltpu
from jax.experimental.pallas import tpu_sc as plsc

N = 10000
E = 160000
NODE_IN = 16
EDGE_IN = 19
OUT = 32
LW = 128

NC = 2
NS = 16
NW = NC * NS


def _leaky(x, s):
    return jnp.where(x >= 0, x, s * x)


def _dot(a, b):
    return jnp.dot(a, b, preferred_element_type=jnp.float32,
                   precision=jax.lax.Precision.HIGHEST)


def _b16(x):
    return x.astype(jnp.bfloat16).astype(jnp.float32)


def _dotb(a, b):
    return jnp.dot(a.astype(jnp.bfloat16), b.astype(jnp.bfloat16),
                   preferred_element_type=jnp.float32)


def _xnorm_body(x_ref, g_ref, b_ref, o_ref):
    xb = x_ref[...]
    m = jnp.mean(xb, axis=0, keepdims=True)
    v = jnp.mean((xb - m) ** 2, axis=0, keepdims=True)
    xn = (xb - m) * jax.lax.rsqrt(v + 1e-5) * g_ref[...] + b_ref[...]
    o_ref[...] = jnp.concatenate(
        [xn, jnp.zeros((N, LW - NODE_IN), jnp.float32)], axis=1)


def _xnorm(x, g, b):
    return pl.pallas_call(
        _xnorm_body,
        out_shape=jax.ShapeDtypeStruct((N, LW), jnp.float32),
    )(x, g.reshape(1, NODE_IN), b.reshape(1, NODE_IN))


_EB = 8000
_ESTEPS = E // _EB


def _estats_body(e_ref, g_ref, b_ref, o_ref, acc_ref):
    step = pl.program_id(0)
    eb = e_ref[...]
    s = jnp.sum(eb, axis=0, keepdims=True)
    ss = jnp.sum(eb * eb, axis=0, keepdims=True)
    blk = jnp.concatenate([s, ss], axis=0)

    @pl.when(step == 0)
    def _():
        acc_ref[...] = blk

    @pl.when(step != 0)
    def _():
        acc_ref[...] += blk

    @pl.when(step == _ESTEPS - 1)
    def _():
        mean = acc_ref[0:1] * (1.0 / E)
        var = acc_ref[1:2] * (1.0 / E) - mean * mean
        scale = jax.lax.rsqrt(var + 1e-5) * g_ref[...]
        shift = b_ref[...] - mean * scale
        o_ref[...] = jnp.concatenate([scale, shift], axis=0)


def _estats(e, g, b):
    return pl.pallas_call(
        _estats_body,
        grid=(_ESTEPS,),
        in_specs=[
            pl.BlockSpec((_EB, EDGE_IN), lambda i: (i, 0)),
            pl.BlockSpec((1, EDGE_IN), lambda i: (0, 0)),
            pl.BlockSpec((1, EDGE_IN), lambda i: (0, 0)),
        ],
        out_specs=pl.BlockSpec((2, EDGE_IN), lambda i: (0, 0)),
        out_shape=jax.ShapeDtypeStruct((2, EDGE_IN), jnp.float32),
        scratch_shapes=[pltpu.VMEM((2, EDGE_IN), jnp.float32)],
    )(e, g.reshape(1, EDGE_IN), b.reshape(1, EDGE_IN))


_BE = 800


def _nnconv_body(fin, e_ref, xs_ref, ss_ref, w1_ref, b1_ref, w2_ref, b2_ref,
                 r_ref, o_ref):
    eb = e_ref[...] * ss_ref[0:1] + ss_ref[1:2]
    h = _dot(eb, w1_ref[...]) + b1_ref[...]
    h = _leaky(h, 0.1)
    h = jnp.dot(h.astype(jnp.bfloat16), w2_ref[...],
                preferred_element_type=jnp.float32) + b2_ref[...]
    h = _leaky(h, 0.1)
    xs = xs_ref[:, :fin]
    if fin == OUT:
        xs = _b16(xs)
        h = _b16(h)
    xe = pltpu.repeat(xs, OUT, axis=1)
    m = _dot(xe * h, r_ref[...])
    o_ref[...] = jnp.concatenate(
        [m, jnp.zeros((m.shape[0], LW - OUT), jnp.float32)], axis=1)


def _reduce_mat_omajor(fin, fout):
    r = np.zeros((fin * fout, fout), np.float32)
    for o in range(fout):
        r[o * fin:(o + 1) * fin, o] = 1.0
    return jnp.asarray(r)


def _nnconv_layer(e, xs, ss, w1, b1, w2, b2, fin):
    k = fin * OUT
    perm = np.arange(k).reshape(fin, OUT).T.reshape(-1)
    w2p = w2[:, perm].astype(jnp.bfloat16)
    b2p = b2[perm]
    r = _reduce_mat_omajor(fin, OUT)
    full = lambda shape: pl.BlockSpec(shape, lambda i: (0,) * len(shape))
    return pl.pallas_call(
        functools.partial(_nnconv_body, fin),
        grid=(E // _BE,),
        in_specs=[
            pl.BlockSpec((_BE, EDGE_IN), lambda i: (i, 0)),
            pl.BlockSpec((_BE, LW), lambda i: (i, 0)),
            full((2, EDGE_IN)),
            full((EDGE_IN, fin)),
            full((1, fin)),
            full((fin, k)),
            full((1, k)),
            full((k, OUT)),
        ],
        out_specs=pl.BlockSpec((_BE, LW), lambda i: (i, 0)),
        out_shape=jax.ShapeDtypeStruct((E, LW), jnp.float32),
    )(e, xs, ss, w1, b1.reshape(1, fin), w2p, b2p.reshape(1, k), r)


def _update_body(fin, parts_ref, cur_ref, root_ref, bias_ref, o_ref):
    agg = parts_ref[0, :N, :OUT] + parts_ref[1, :N, :OUT]
    cur = cur_ref[:, :fin]
    if fin == OUT:
        rd = _dotb(cur, root_ref[...])
    else:
        rd = _dot(cur, root_ref[...])
    new = agg + rd + bias_ref[...]
    o_ref[...] = jnp.concatenate(
        [new, jnp.zeros((N, LW - OUT), jnp.float32)], axis=1)


def _update(parts, cur, root, bias, fin):
    return pl.pallas_call(
        functools.partial(_update_body, fin),
        out_shape=jax.ShapeDtypeStruct((N, LW), jnp.float32),
    )(parts, cur, root, bias.reshape(1, OUT))


_BM = 1600


def _edge_mlp_body(e_ref, xs_ref, xd_ref, ss_ref, ws_ref, wd_ref, we_ref,
                   b1_ref, w2_ref, b2_ref, w3_ref, b3_ref, o_ref):
    eb = e_ref[...] * ss_ref[0:1] + ss_ref[1:2]
    xd = xd_ref[:, :OUT]
    h = (_dotb(xs_ref[:, :OUT], ws_ref[...])
         + _dotb(xd, wd_ref[...])
         + _dotb(eb, we_ref[...])
         + b1_ref[...])
    h = _leaky(h, 0.1)
    h = _dotb(h, w2_ref[...]) + b2_ref[...]
    h = _leaky(h, 0.1)
    t = _dotb(h, w3_ref[...]) + b3_ref[...]
    o_ref[...] = jnp.concatenate(
        [xd, t, jnp.zeros((xd.shape[0], LW - OUT - 16), jnp.float32)], axis=1)


def _edge_mlp(e, xs, xd, ss, em_w1, em_b1, em_w2, em_b2, em_w3, em_b3):
    ws = em_w1[0:OUT]
    wd = em_w1[OUT:2 * OUT]
    we = em_w1[2 * OUT:]
    w3p = jnp.zeros((16, 16), jnp.float32).at[:, 0:2].set(em_w3)
    b3p = jnp.zeros((1, 16), jnp.float32).at[0, 0:2].set(em_b3).at[0, 2].set(1.0)
    full = lambda shape: pl.BlockSpec(shape, lambda i: (0,) * len(shape))
    return pl.pallas_call(
        _edge_mlp_body,
        grid=(E // _BM,),
        in_specs=[
            pl.BlockSpec((_BM, EDGE_IN), lambda i: (i, 0)),
            pl.BlockSpec((_BM, LW), lambda i: (i, 0)),
            pl.BlockSpec((_BM, LW), lambda i: (i, 0)),
            full((2, EDGE_IN)),
            full((OUT, 64)),
            full((OUT, 64)),
            full((EDGE_IN, 64)),
            full((1, 64)),
            full((64, 16)),
            full((1, 16)),
            full((16, 16)),
            full((1, 16)),
        ],
        out_specs=pl.BlockSpec((_BM, LW), lambda i: (i, 0)),
        out_shape=jax.ShapeDtypeStruct((E, LW), jnp.float32),
    )(e, xs, xd, ss, ws, wd, we, em_b1.reshape(1, 64), em_w2,
      em_b2.reshape(1, 16), w3p, b3p)


def _final_body(parts_ref, sel_ref, w1_ref, b1_ref, w2_ref, b2_ref, o_ref):
    s = parts_ref[0, :N] + parts_ref[1, :N]
    cnt = _dot(s, sel_ref[...])
    mean = s / jnp.maximum(cnt, 1.0)
    nh = _leaky(_dotb(mean, w1_ref[...])
                + b1_ref[...], 0.12)
    logits = _dotb(nh, w2_ref[...]) + b2_ref[...]
    m = jnp.max(logits, axis=1, keepdims=True)
    shifted = logits - m
    o_ref[...] = shifted - jnp.log(jnp.sum(jnp.exp(shifted), axis=1, keepdims=True))


def _final(parts, nm_w1, nm_b1, nm_w2, nm_b2):
    sel = jnp.zeros((LW, 1), jnp.float32).at[OUT + 2, 0].set(1.0)
    w1p = jnp.zeros((LW, 16), jnp.float32).at[0:OUT + 2].set(nm_w1)
    return pl.pallas_call(
        _final_body,
        out_shape=jax.ShapeDtypeStruct((N, 2), jnp.float32),
    )(parts, sel, w1p, nm_b1.reshape(1, 16), nm_w2, nm_b2.reshape(1, 2))


_GCHUNK = 200


def _gather_rows(table, idx):
    per = E // NW
    nchunks = per // _GCHUNK
    mesh = plsc.VectorSubcoreMesh(core_axis_name="c", subcore_axis_name="s")

    @functools.partial(
        pl.kernel,
        out_type=jax.ShapeDtypeStruct((E, LW), jnp.float32),
        mesh=mesh,
        scratch_types=[
            pltpu.VMEM((_GCHUNK,), jnp.int32),
            pltpu.VMEM((_GCHUNK, LW), jnp.float32),
            pltpu.SemaphoreType.DMA,
        ],
    )
    def gk(table_hbm, idx_hbm, out_hbm, idx_v, rows_v, sem):
        wid = lax.axis_index("s") * NC + lax.axis_index("c")
        base = wid * per

        def body(j, carry):
            off = base + j * _GCHUNK
            pltpu.sync_copy(idx_hbm.at[pl.ds(off, _GCHUNK)], idx_v)
            pltpu.async_copy(table_hbm.at[idx_v], rows_v, sem).wait()
            pltpu.sync_copy(rows_v, out_hbm.at[pl.ds(off, _GCHUNK)])
            return carry

        lax.fori_loop(0, nchunks, body, 0)

    return gk(table, idx)


_SCHUNK = 200
NP = 10240


def _scatter_add(data, idx):
    per_core = E // NC
    per = E // NW
    nchunks = per // _SCHUNK
    rows_per_tile = NP // NS
    zeros = jnp.zeros((NP, LW), jnp.float32)
    mesh = plsc.VectorSubcoreMesh(core_axis_name="c", subcore_axis_name="s")

    @functools.partial(
        pl.kernel,
        out_type=jax.ShapeDtypeStruct((NC * NP, LW), jnp.float32),
        mesh=mesh,
        scratch_types=[
            pltpu.VMEM((_SCHUNK,), jnp.int32),
            pltpu.VMEM((_SCHUNK, LW), jnp.float32),
            pltpu.VMEM_SHARED((NP, LW), jnp.float32),
        ],
    )
    def sk(data_hbm, idx_hbm, zeros_hbm, out_hbm, idx_v, dat_v, acc):
        cid = lax.axis_index("c")
        sid = lax.axis_index("s")
        r0 = sid * rows_per_tile
        pltpu.sync_copy(zeros_hbm.at[pl.ds(r0, rows_per_tile)],
                        acc.at[pl.ds(r0, rows_per_tile)])
        plsc.subcore_barrier()
        base = cid * per_core + sid * per

        def body(j, carry):
            off = base + j * _SCHUNK
            pltpu.sync_copy(idx_hbm.at[pl.ds(off, _SCHUNK)], idx_v)
            pltpu.sync_copy(data_hbm.at[pl.ds(off, _SCHUNK)], dat_v)
            pltpu.sync_copy(dat_v, acc.at[idx_v], add=True)
            return carry

        lax.fori_loop(0, nchunks, body, 0)
        plsc.subcore_barrier()
        pltpu.sync_copy(acc.at[pl.ds(r0, rows_per_tile)],
                        out_hbm.at[pl.ds(cid * NP + r0, rows_per_tile)])

    return sk(data, idx, zeros).reshape(NC, NP, LW)


def kernel(x, edge_index, e, xbatch, bn_node_g, bn_node_b, bn_edge_g, bn_edge_b,
           nn0_w1, nn0_b1, nn0_w2, nn0_b2, root0, bias0,
           nn1_w1, nn1_b1, nn1_w2, nn1_b2, root1, bias1,
           nn2_w1, nn2_b1, nn2_w2, nn2_b2, root2, bias2,
           em_w1, em_b1, em_w2, em_b2, em_w3, em_b3,
           nm_w1, nm_b1, nm_w2, nm_b2):
    src = edge_index[0].astype(jnp.int32)
    dst = edge_index[1].astype(jnp.int32)

    cur = _xnorm(x, bn_node_g, bn_node_b)
    ss = _estats(e, bn_edge_g, bn_edge_b)

    layers = [
        (nn0_w1, nn0_b1, nn0_w2, nn0_b2, root0, bias0, NODE_IN),
        (nn1_w1, nn1_b1, nn1_w2, nn1_b2, root1, bias1, OUT),
        (nn2_w1, nn2_b1, nn2_w2, nn2_b2, root2, bias2, OUT),
    ]
    for w1, b1, w2, b2, root, bias, fin in layers:
        xs = _gather_rows(cur, src)
        msg = _nnconv_layer(e, xs, ss, w1, b1, w2, b2, fin)
        parts = _scatter_add(msg, dst)
        cur = _update(parts, cur, root, bias, fin)

    xs = _gather_rows(cur, src)
    xd = _gather_rows(cur, dst)
    nm = _edge_mlp(e, xs, xd, ss, em_w1, em_b1, em_w2, em_b2, em_w3, em_b3)
    parts = _scatter_add(nm, src)
    return _final(parts, nm_w1, nm_b1, nm_w2, nm_b2)


# speedup vs baseline: 1.5261x; 1.0128x over previous
import functools

import numpy as np
import jax
import jax.numpy as jnp
from jax import lax
from jax.experimental import pallas as pl
from jax.experimental.pallas import tpu as pltpu
from jax.experimental.pallas import tpu_sc as plsc

N = 10000
E = 160000
NODE_IN = 16
EDGE_IN = 19
OUT = 32
LW = 128

NC = 2
NS = 16
NW = NC * NS


def _leaky(x, s):
    return jnp.where(x >= 0, x, s * x)


def _dot(a, b):
    return jnp.dot(a, b, preferred_element_type=jnp.float32,
                   precision=jax.lax.Precision.HIGHEST)


def _b16(x):
    return x.astype(jnp.bfloat16).astype(jnp.float32)


def _dotb(a, b):
    return jnp.dot(a.astype(jnp.bfloat16), b.astype(jnp.bfloat16),
                   preferred_element_type=jnp.float32)


def _xnorm_body(x_ref, g_ref, b_ref, o_ref):
    xb = x_ref[...]
    m = jnp.mean(xb, axis=0, keepdims=True)
    v = jnp.mean((xb - m) ** 2, axis=0, keepdims=True)
    xn = (xb - m) * jax.lax.rsqrt(v + 1e-5) * g_ref[...] + b_ref[...]
    o_ref[...] = jnp.concatenate(
        [xn, jnp.zeros((N, LW - NODE_IN), jnp.float32)], axis=1)


def _xnorm(x, g, b):
    return pl.pallas_call(
        _xnorm_body,
        out_shape=jax.ShapeDtypeStruct((N, LW), jnp.float32),
    )(x, g.reshape(1, NODE_IN), b.reshape(1, NODE_IN))


_EB = 8000
_ESTEPS = E // _EB


def _estats_body(e_ref, g_ref, b_ref, o_ref, acc_ref):
    step = pl.program_id(0)
    eb = e_ref[...]
    s = jnp.sum(eb, axis=0, keepdims=True)
    ss = jnp.sum(eb * eb, axis=0, keepdims=True)
    blk = jnp.concatenate([s, ss], axis=0)

    @pl.when(step == 0)
    def _():
        acc_ref[...] = blk

    @pl.when(step != 0)
    def _():
        acc_ref[...] += blk

    @pl.when(step == _ESTEPS - 1)
    def _():
        mean = acc_ref[0:1] * (1.0 / E)
        var = acc_ref[1:2] * (1.0 / E) - mean * mean
        scale = jax.lax.rsqrt(var + 1e-5) * g_ref[...]
        shift = b_ref[...] - mean * scale
        o_ref[...] = jnp.concatenate([scale, shift], axis=0)


def _estats(e, g, b):
    return pl.pallas_call(
        _estats_body,
        grid=(_ESTEPS,),
        in_specs=[
            pl.BlockSpec((_EB, EDGE_IN), lambda i: (i, 0)),
            pl.BlockSpec((1, EDGE_IN), lambda i: (0, 0)),
            pl.BlockSpec((1, EDGE_IN), lambda i: (0, 0)),
        ],
        out_specs=pl.BlockSpec((2, EDGE_IN), lambda i: (0, 0)),
        out_shape=jax.ShapeDtypeStruct((2, EDGE_IN), jnp.float32),
        scratch_shapes=[pltpu.VMEM((2, EDGE_IN), jnp.float32)],
    )(e, g.reshape(1, EDGE_IN), b.reshape(1, EDGE_IN))


_BE = 800


def _nnconv_body(fin, e_ref, xs_ref, ss_ref, w1_ref, b1_ref, w2_ref, b2_ref,
                 r_ref, o_ref):
    eb = e_ref[...] * ss_ref[0:1] + ss_ref[1:2]
    h = _dot(eb, w1_ref[...]) + b1_ref[...]
    h = _leaky(h, 0.1)
    h = jnp.dot(h.astype(jnp.bfloat16), w2_ref[...],
                preferred_element_type=jnp.float32) + b2_ref[...]
    h = _leaky(h, 0.1)
    xs = xs_ref[:, :fin]
    if fin == OUT:
        xs = _b16(xs)
        h = _b16(h)
    xe = pltpu.repeat(xs, OUT, axis=1)
    m = _dot(xe * h, r_ref[...])
    o_ref[...] = jnp.concatenate(
        [m, jnp.zeros((m.shape[0], LW - OUT), jnp.float32)], axis=1)


def _reduce_mat_omajor(fin, fout):
    r = np.zeros((fin * fout, fout), np.float32)
    for o in range(fout):
        r[o * fin:(o + 1) * fin, o] = 1.0
    return jnp.asarray(r)


def _nnconv_layer(e, xs, ss, w1, b1, w2, b2, fin):
    k = fin * OUT
    perm = np.arange(k).reshape(fin, OUT).T.reshape(-1)
    w2p = w2[:, perm].astype(jnp.bfloat16)
    b2p = b2[perm]
    r = _reduce_mat_omajor(fin, OUT)
    full = lambda shape: pl.BlockSpec(shape, lambda i: (0,) * len(shape))
    return pl.pallas_call(
        functools.partial(_nnconv_body, fin),
        grid=(E // _BE,),
        in_specs=[
            pl.BlockSpec((_BE, EDGE_IN), lambda i: (i, 0)),
            pl.BlockSpec((_BE, LW), lambda i: (i, 0)),
            full((2, EDGE_IN)),
            full((EDGE_IN, fin)),
            full((1, fin)),
            full((fin, k)),
            full((1, k)),
            full((k, OUT)),
        ],
        out_specs=pl.BlockSpec((_BE, LW), lambda i: (i, 0)),
        out_shape=jax.ShapeDtypeStruct((E, LW), jnp.float32),
    )(e, xs, ss, w1, b1.reshape(1, fin), w2p, b2p.reshape(1, k), r)


def _update_body(fin, parts_ref, cur_ref, root_ref, bias_ref, o_ref):
    agg = parts_ref[0, :N, :OUT] + parts_ref[1, :N, :OUT]
    cur = cur_ref[:, :fin]
    if fin == OUT:
        rd = _dotb(cur, root_ref[...])
    else:
        rd = _dot(cur, root_ref[...])
    new = agg + rd + bias_ref[...]
    o_ref[...] = jnp.concatenate(
        [new, jnp.zeros((N, LW - OUT), jnp.float32)], axis=1)


def _update(parts, cur, root, bias, fin):
    return pl.pallas_call(
        functools.partial(_update_body, fin),
        out_shape=jax.ShapeDtypeStruct((N, LW), jnp.float32),
    )(parts, cur, root, bias.reshape(1, OUT))


_BM = 1600


def _edge_mlp_body(e_ref, xs_ref, xd_ref, ss_ref, ws_ref, wd_ref, we_ref,
                   b1_ref, w2_ref, b2_ref, w3_ref, b3_ref, o_ref):
    eb = e_ref[...] * ss_ref[0:1] + ss_ref[1:2]
    xd = xd_ref[:, :OUT]
    h = (_dotb(xs_ref[:, :OUT], ws_ref[...])
         + _dotb(xd, wd_ref[...])
         + _dotb(eb, we_ref[...])
         + b1_ref[...])
    h = _leaky(h, 0.1)
    h = _dotb(h, w2_ref[...]) + b2_ref[...]
    h = _leaky(h, 0.1)
    t = _dotb(h, w3_ref[...]) + b3_ref[...]
    o_ref[...] = jnp.concatenate(
        [xd, t, jnp.zeros((xd.shape[0], LW - OUT - 16), jnp.float32)], axis=1)


def _edge_mlp(e, xs, xd, ss, em_w1, em_b1, em_w2, em_b2, em_w3, em_b3):
    ws = em_w1[0:OUT]
    wd = em_w1[OUT:2 * OUT]
    we = em_w1[2 * OUT:]
    w3p = jnp.zeros((16, 16), jnp.float32).at[:, 0:2].set(em_w3)
    b3p = jnp.zeros((1, 16), jnp.float32).at[0, 0:2].set(em_b3).at[0, 2].set(1.0)
    full = lambda shape: pl.BlockSpec(shape, lambda i: (0,) * len(shape))
    return pl.pallas_call(
        _edge_mlp_body,
        grid=(E // _BM,),
        in_specs=[
            pl.BlockSpec((_BM, EDGE_IN), lambda i: (i, 0)),
            pl.BlockSpec((_BM, LW), lambda i: (i, 0)),
            pl.BlockSpec((_BM, LW), lambda i: (i, 0)),
            full((2, EDGE_IN)),
            full((OUT, 64)),
            full((OUT, 64)),
            full((EDGE_IN, 64)),
            full((1, 64)),
            full((64, 16)),
            full((1, 16)),
            full((16, 16)),
            full((1, 16)),
        ],
        out_specs=pl.BlockSpec((_BM, LW), lambda i: (i, 0)),
        out_shape=jax.ShapeDtypeStruct((E, LW), jnp.float32),
    )(e, xs, xd, ss, ws, wd, we, em_b1.reshape(1, 64), em_w2,
      em_b2.reshape(1, 16), w3p, b3p)


def _final_body(parts_ref, sel_ref, w1_ref, b1_ref, w2_ref, b2_ref, o_ref):
    s = parts_ref[0, :N] + parts_ref[1, :N]
    cnt = _dot(s, sel_ref[...])
    mean = s / jnp.maximum(cnt, 1.0)
    nh = _leaky(_dotb(mean, w1_ref[...])
                + b1_ref[...], 0.12)
    logits = _dotb(nh, w2_ref[...]) + b2_ref[...]
    m = jnp.max(logits, axis=1, keepdims=True)
    shifted = logits - m
    o_ref[...] = shifted - jnp.log(jnp.sum(jnp.exp(shifted), axis=1, keepdims=True))


def _final(parts, nm_w1, nm_b1, nm_w2, nm_b2):
    sel = jnp.zeros((LW, 1), jnp.float32).at[OUT + 2, 0].set(1.0)
    w1p = jnp.zeros((LW, 16), jnp.float32).at[0:OUT + 2].set(nm_w1)
    return pl.pallas_call(
        _final_body,
        out_shape=jax.ShapeDtypeStruct((N, 2), jnp.float32),
    )(parts, sel, w1p, nm_b1.reshape(1, 16), nm_w2, nm_b2.reshape(1, 2))


_GCHUNK = 200


def _gather_rows(table, idx):
    per = E // NW
    nchunks = per // _GCHUNK
    mesh = plsc.VectorSubcoreMesh(core_axis_name="c", subcore_axis_name="s")

    @functools.partial(
        pl.kernel,
        out_type=jax.ShapeDtypeStruct((E, LW), jnp.float32),
        mesh=mesh,
        scratch_types=[
            pltpu.VMEM((_GCHUNK,), jnp.int32),
            pltpu.VMEM((_GCHUNK, LW), jnp.float32),
            pltpu.SemaphoreType.DMA,
        ],
    )
    def gk(table_hbm, idx_hbm, out_hbm, idx_v, rows_v, sem):
        wid = lax.axis_index("s") * NC + lax.axis_index("c")
        base = wid * per

        def body(j, carry):
            off = base + j * _GCHUNK
            pltpu.sync_copy(idx_hbm.at[pl.ds(off, _GCHUNK)], idx_v)
            pltpu.async_copy(table_hbm.at[idx_v], rows_v, sem).wait()
            pltpu.sync_copy(rows_v, out_hbm.at[pl.ds(off, _GCHUNK)])
            return carry

        lax.fori_loop(0, nchunks, body, 0)

    return gk(table, idx)


_SCHUNK = 200
NP = 10240


def _scatter_add(data, idx):
    per_core = E // NC
    per = E // NW
    nchunks = per // _SCHUNK
    rows_per_tile = NP // NS
    zeros = jnp.zeros((NP, LW), jnp.float32)
    mesh = plsc.VectorSubcoreMesh(core_axis_name="c", subcore_axis_name="s")

    @functools.partial(
        pl.kernel,
        out_type=jax.ShapeDtypeStruct((NC * NP, LW), jnp.float32),
        mesh=mesh,
        scratch_types=[
            pltpu.VMEM((_SCHUNK,), jnp.int32),
            pltpu.VMEM((_SCHUNK, LW), jnp.float32),
            pltpu.VMEM_SHARED((NP, LW), jnp.float32),
            pltpu.SemaphoreType.DMA,
            pltpu.SemaphoreType.DMA,
        ],
    )
    def sk(data_hbm, idx_hbm, zeros_hbm, out_hbm, idx_v, dat_v, acc, sem1, sem2):
        cid = lax.axis_index("c")
        sid = lax.axis_index("s")
        r0 = sid * rows_per_tile
        pltpu.sync_copy(zeros_hbm.at[pl.ds(r0, rows_per_tile)],
                        acc.at[pl.ds(r0, rows_per_tile)])
        plsc.subcore_barrier()
        base = cid * per_core + sid * per

        def body(j, carry):
            off = base + j * _SCHUNK
            c1 = pltpu.async_copy(idx_hbm.at[pl.ds(off, _SCHUNK)], idx_v, sem1)
            c2 = pltpu.async_copy(data_hbm.at[pl.ds(off, _SCHUNK)], dat_v, sem2)
            c1.wait()
            c2.wait()
            pltpu.sync_copy(dat_v, acc.at[idx_v], add=True)
            return carry

        lax.fori_loop(0, nchunks, body, 0)
        plsc.subcore_barrier()
        pltpu.sync_copy(acc.at[pl.ds(r0, rows_per_tile)],
                        out_hbm.at[pl.ds(cid * NP + r0, rows_per_tile)])

    return sk(data, idx, zeros).reshape(NC, NP, LW)


def kernel(x, edge_index, e, xbatch, bn_node_g, bn_node_b, bn_edge_g, bn_edge_b,
           nn0_w1, nn0_b1, nn0_w2, nn0_b2, root0, bias0,
           nn1_w1, nn1_b1, nn1_w2, nn1_b2, root1, bias1,
           nn2_w1, nn2_b1, nn2_w2, nn2_b2, root2, bias2,
           em_w1, em_b1, em_w2, em_b2, em_w3, em_b3,
           nm_w1, nm_b1, nm_w2, nm_b2):
    src = edge_index[0].astype(jnp.int32)
    dst = edge_index[1].astype(jnp.int32)

    cur = _xnorm(x, bn_node_g, bn_node_b)
    ss = _estats(e, bn_edge_g, bn_edge_b)

    layers = [
        (nn0_w1, nn0_b1, nn0_w2, nn0_b2, root0, bias0, NODE_IN),
        (nn1_w1, nn1_b1, nn1_w2, nn1_b2, root1, bias1, OUT),
        (nn2_w1, nn2_b1, nn2_w2, nn2_b2, root2, bias2, OUT),
    ]
    for w1, b1, w2, b2, root, bias, fin in layers:
        xs = _gather_rows(cur, src)
        msg = _nnconv_layer(e, xs, ss, w1, b1, w2, b2, fin)
        parts = _scatter_add(msg, dst)
        cur = _update(parts, cur, root, bias, fin)

    xs = _gather_rows(cur, src)
    xd = _gather_rows(cur, dst)
    nm = _edge_mlp(e, xs, xd, ss, em_w1, em_b1, em_w2, em_b2, em_w3, em_b3)
    parts = _scatter_add(nm, src)
    return _final(parts, nm_w1, nm_b1, nm_w2, nm_b2)
